# baseline (speedup 1.0000x reference)
import functools

import jax
import jax.numpy as jnp
from jax import lax
from jax.experimental import pallas as pl
from jax.experimental.pallas import tpu as pltpu
from jax.experimental.pallas import tpu_sc as plsc

_N = 10000
_E = 160000
_D = 128
_NW = 32
_CH = 128
_NCHUNK = _E // _CH
_CPW = _NCHUNK // _NW
_NEXTRA = _NCHUNK - _CPW * _NW
_WEXTRA = _NW - _NEXTRA
_TRIPLES = _CPW // 3
_ZR = 16
_NG = _N // _ZR


def _chunk_base(wid):
    return jnp.where(wid < _WEXTRA, _CPW * wid,
                     _CPW * _WEXTRA + (_CPW + 1) * (wid - _WEXTRA))


def _sc_mesh():
    return plsc.VectorSubcoreMesh(core_axis_name="c", subcore_axis_name="s")


def _gather_pairs(x, src3, dst3):
    @functools.partial(
        pl.kernel,
        out_type=(jax.ShapeDtypeStruct((_E, _D), jnp.float32),
                  jax.ShapeDtypeStruct((_E, _D), jnp.float32)),
        mesh=_sc_mesh(),
        scratch_types=[
            pltpu.VMEM((_CPW + 1, 1, _CH), jnp.int32),
            pltpu.VMEM((_CPW + 1, 1, _CH), jnp.int32),
            pltpu.VMEM((3, _CH, _D), jnp.float32),
            pltpu.VMEM((3, _CH, _D), jnp.float32),
            pltpu.SemaphoreType.DMA,
            pltpu.SemaphoreType.DMA,
            pltpu.SemaphoreType.DMA,
            pltpu.SemaphoreType.DMA,
            pltpu.SemaphoreType.DMA,
            pltpu.SemaphoreType.DMA,
        ],
    )
    def k(x_hbm, src_hbm, dst_hbm, xs_hbm, xd_hbm,
          isv, idv, rs, rd, sg0, sg1, sg2, sw0, sw1, sw2):
        cid = lax.axis_index("c")
        sid = lax.axis_index("s")
        wid = sid * 2 + cid
        cb = _chunk_base(wid)
        extra = wid >= _WEXTRA
        semg = (sg0, sg1, sg2)
        semw = (sw0, sw1, sw2)

        pltpu.sync_copy(src_hbm.at[pl.ds(cb, _CPW)], isv.at[pl.ds(0, _CPW)])
        pltpu.sync_copy(dst_hbm.at[pl.ds(cb, _CPW)], idv.at[pl.ds(0, _CPW)])

        @pl.when(extra)
        def _():
            pltpu.sync_copy(src_hbm.at[pl.ds(cb + _CPW, 1)],
                            isv.at[pl.ds(_CPW, 1)])
            pltpu.sync_copy(dst_hbm.at[pl.ds(cb + _CPW, 1)],
                            idv.at[pl.ds(_CPW, 1)])

        def drain_wb(k_):
            pltpu.make_async_copy(rs.at[k_], xs_hbm.at[pl.ds(0, _CH)],
                                  semw[k_]).wait()
            pltpu.make_async_copy(rd.at[k_], xd_hbm.at[pl.ds(0, _CH)],
                                  semw[k_]).wait()

        def gath(t, k_):
            gs = pltpu.async_copy(x_hbm.at[isv.at[t, 0]], rs.at[k_], semg[k_])
            gd = pltpu.async_copy(x_hbm.at[idv.at[t, 0]], rd.at[k_], semg[k_])
            return gs, gd

        def wback(t, k_):
            off = pl.multiple_of((cb + t) * _CH, 8)
            pltpu.async_copy(rs.at[k_], xs_hbm.at[pl.ds(off, _CH)], semw[k_])
            pltpu.async_copy(rd.at[k_], xd_hbm.at[pl.ds(off, _CH)], semw[k_])

        def triple(q, carry):
            t0 = q * 3
            hs = []
            for k_ in range(3):
                @pl.when(q > 0)
                def _(k_=k_):
                    drain_wb(k_)
                hs.append(gath(t0 + k_, k_))
            for k_ in range(3):
                gs, gd = hs[k_]
                gs.wait()
                gd.wait()
                wback(t0 + k_, k_)
            return carry

        lax.fori_loop(0, _TRIPLES, triple, 0)

        @pl.when(extra)
        def _():
            drain_wb(0)
            gs, gd = gath(_CPW, 0)
            gs.wait()
            gd.wait()
            wback(_CPW, 0)

        for k_ in range(3):
            drain_wb(k_)

    return k(x, src3, dst3)


def _segment_partials(e, dst3):
    @functools.partial(
        pl.kernel,
        out_type=jax.ShapeDtypeStruct((2, _N, _D), jnp.float32),
        mesh=_sc_mesh(),
        scratch_types=[
            pltpu.VMEM((_CPW + 1, 1, _CH), jnp.int32),
            pltpu.VMEM((2, _CH, _D), jnp.float32),
            pltpu.VMEM((_ZR, _D), jnp.float32),
            pltpu.VMEM_SHARED((_N, _D), jnp.float32),
            pltpu.SemaphoreType.DMA,
            pltpu.SemaphoreType.DMA,
            pltpu.SemaphoreType.DMA,
            pltpu.SemaphoreType.DMA,
        ],
    )
    def k(e_hbm, dst_hbm, out_hbm, idv, rw, zbuf, acc,
          sl0, sl1, ss0, ss1):
        cid = lax.axis_index("c")
        sid = lax.axis_index("s")
        wid = sid * 2 + cid
        cb = _chunk_base(wid)
        extra = wid >= _WEXTRA
        seml = (sl0, sl1)
        sems = (ss0, ss1)

        zero = jnp.zeros((16,), jnp.float32)
        for r in range(_ZR):
            for l in range(_D // 16):
                zbuf[r, pl.ds(16 * l, 16)] = zero
        ngrp = jnp.where(sid < _NG - 16 * (_NG // 16), _NG // 16 + 1, _NG // 16)

        def zfill(gk, carry):
            roff = pl.multiple_of((sid + gk * 16) * _ZR, 8)
            pltpu.sync_copy(zbuf, acc.at[pl.ds(roff, _ZR)])
            return carry

        lax.fori_loop(0, ngrp, zfill, 0)

        pltpu.sync_copy(dst_hbm.at[pl.ds(cb, _CPW)], idv.at[pl.ds(0, _CPW)])

        @pl.when(extra)
        def _():
            pltpu.sync_copy(dst_hbm.at[pl.ds(cb + _CPW, 1)],
                            idv.at[pl.ds(_CPW, 1)])

        plsc.subcore_barrier()

        def drain_add(k_):
            pltpu.make_async_copy(rw.at[k_], acc.at[pl.ds(0, _CH)],
                                  sems[k_]).wait()

        def load(t, k_):
            off = pl.multiple_of((cb + t) * _CH, 8)
            return pltpu.async_copy(e_hbm.at[pl.ds(off, _CH)], rw.at[k_],
                                    seml[k_])

        def spair(p, carry):
            t0 = p * 2
            hs = []
            for k_ in range(2):
                @pl.when(p > 0)
                def _(k_=k_):
                    drain_add(k_)
                hs.append(load(t0 + k_, k_))
            for k_ in range(2):
                hs[k_].wait()
                pltpu.async_copy(rw.at[k_], acc.at[idv.at[t0 + k_, 0]],
                                 sems[k_], add=True)
            return carry

        lax.fori_loop(0, (_CPW - 1) // 2, spair, 0)

        drain_add(0)
        l0 = load(_CPW - 1, 0)
        l0.wait()
        pltpu.async_copy(rw.at[0], acc.at[idv.at[_CPW - 1, 0]],
                         sems[0], add=True)

        @pl.when(extra)
        def _():
            drain_add(1)
            l1 = load(_CPW, 1)
            l1.wait()
            pltpu.async_copy(rw.at[1], acc.at[idv.at[_CPW, 0]],
                             sems[1], add=True)

        drain_add(0)
        drain_add(1)
        plsc.subcore_barrier()

        def wback(gk, carry):
            roff = pl.multiple_of((sid + gk * 16) * _ZR, 8)
            pltpu.sync_copy(acc.at[pl.ds(roff, _ZR)],
                            out_hbm.at[cid, pl.ds(roff, _ZR)])
            return carry

        lax.fori_loop(0, ngrp, wback, 0)

    return k(e, dst3)


def _mlp_pallas(groups, w1s, b1, w2, b2, w3, b3, ln_g, ln_b, resid_gi, br):
    flat = [a for g in groups for a in g]
    sizes = [len(g) for g in groups]
    n = flat[0].shape[0]
    num_in = len(flat)
    ng = len(w1s)
    has_ln = ln_g is not None

    row_spec = lambda d: pl.BlockSpec((br, d), lambda i: (i, 0))
    full_spec = lambda s: pl.BlockSpec(s, lambda i: (0, 0))

    in_specs = [row_spec(a.shape[1]) for a in flat]
    in_specs += [full_spec(w.shape) for w in w1s]
    b1r = b1.reshape(1, _D)
    b2r = b2.reshape(1, _D)
    b3r = b3.reshape(1, _D)
    extras = [b1r, w2, b2r, w3, b3r]
    in_specs += [full_spec(b1r.shape), full_spec(w2.shape),
                 full_spec(b2r.shape), full_spec(w3.shape),
                 full_spec(b3r.shape)]
    if has_ln:
        extras += [ln_g.reshape(1, _D), ln_b.reshape(1, _D)]
        in_specs += [full_spec((1, _D)), full_spec((1, _D))]

    def body(*refs):
        irefs = refs[:num_in]
        wrefs = refs[num_in:num_in + ng]
        rest = refs[num_in + ng:]
        b1_r, w2_r, b2_r, w3_r, b3_r = rest[:5]
        out_ref = refs[-1]
        pos = 0
        h = None
        res = None
        for gi, sz in enumerate(sizes):
            xg = irefs[pos][...]
            for j in range(1, sz):
                xg = xg + irefs[pos + j][...]
            t = jnp.dot(xg, wrefs[gi][...], preferred_element_type=jnp.float32)
            h = t if h is None else h + t
            if resid_gi is not None and gi == resid_gi:
                res = xg
            pos += sz
        h = jax.nn.relu(h + b1_r[...])
        h = jax.nn.relu(jnp.dot(h, w2_r[...], preferred_element_type=jnp.float32) + b2_r[...])
        h = jnp.dot(h, w3_r[...], preferred_element_type=jnp.float32) + b3_r[...]
        if has_ln:
            g_r, bb_r = rest[5], rest[6]
            mu = jnp.mean(h, axis=-1, keepdims=True)
            var = jnp.mean((h - mu) * (h - mu), axis=-1, keepdims=True)
            h = (h - mu) * lax.rsqrt(var + 1e-5) * g_r[...] + bb_r[...]
        if res is not None:
            h = res + h
        out_ref[...] = h

    return pl.pallas_call(
        body,
        grid=(n // br,),
        in_specs=in_specs,
        out_specs=pl.BlockSpec((br, _D), lambda i: (i, 0)),
        out_shape=jax.ShapeDtypeStruct((n, _D), jnp.float32),
    )(*flat, *w1s, *extras)


def kernel(init_position, time_vector, node_type, edge_index, edge_features, params):
    p = params
    onehot = jax.nn.one_hot(node_type, 9, dtype=jnp.float32)
    nf = jnp.concatenate(
        [init_position, time_vector[:, None], onehot,
         jnp.zeros((_N, 4), jnp.float32)], axis=1)
    ef = jnp.concatenate(
        [edge_features, jnp.zeros((_E, 5), jnp.float32)], axis=1)
    src3 = edge_index[0].reshape(_NCHUNK, 1, _CH)
    dst3 = edge_index[1].reshape(_NCHUNK, 1, _CH)

    ne = p["node_enc"]
    w1n = jnp.concatenate([ne["W"][0], jnp.zeros((4, _D), jnp.float32)], axis=0)
    x = _mlp_pallas([[nf]], [w1n], ne["b"][0], ne["W"][1], ne["b"][1],
                    ne["W"][2], ne["b"][2], ne["ln_g"], ne["ln_b"],
                    resid_gi=None, br=1000)

    ee = p["edge_enc"]
    w1e = jnp.concatenate([ee["W"][0], jnp.zeros((5, _D), jnp.float32)], axis=0)
    e = _mlp_pallas([[ef]], [w1e], ee["b"][0], ee["W"][1], ee["b"][1],
                    ee["W"][2], ee["b"][2], ee["ln_g"], ee["ln_b"],
                    resid_gi=None, br=1000)

    for sp in p["proc"]:
        xs, xd = _gather_pairs(x, src3, dst3)
        ew = sp["edge"]
        wa = ew["W"][0][:_D]
        wb = ew["W"][0][_D:2 * _D]
        wc = ew["W"][0][2 * _D:]
        e = _mlp_pallas([[xs], [xd], [e]], [wa, wb, wc], ew["b"][0],
                        ew["W"][1], ew["b"][1], ew["W"][2], ew["b"][2],
                        ew["ln_g"], ew["ln_b"], resid_gi=2, br=1000)
        parts = _segment_partials(e, dst3)
        nw = sp["node"]
        na = nw["W"][0][:_D]
        nb = nw["W"][0][_D:]
        x = _mlp_pallas([[x], [parts[0], parts[1]]], [na, nb], nw["b"][0],
                        nw["W"][1], nw["b"][1], nw["W"][2], nw["b"][2],
                        nw["ln_g"], nw["ln_b"], resid_gi=0, br=1000)

    dp = p["dec"]
    w3d = jnp.pad(dp["W"][2], ((0, 0), (0, _D - 2)))
    b3d = jnp.pad(dp["b"][2], (0, _D - 2))
    out = _mlp_pallas([[x]], [dp["W"][0]], dp["b"][0], dp["W"][1], dp["b"][1],
                      w3d, b3d, None, None, resid_gi=None, br=1000)
    return init_position + out[:, :2]


# speedup vs baseline: 1.1235x; 1.1235x over previous
import functools

import jax
import jax.numpy as jnp
from jax import lax
from jax.experimental import pallas as pl
from jax.experimental.pallas import tpu as pltpu
from jax.experimental.pallas import tpu_sc as plsc

_N = 10000
_E = 160000
_D = 128
_NW = 32
_CH = 128
_NCHUNK = _E // _CH
_HALF = _NCHUNK // 2
_EH = _HALF * _CH
_ZR = 16
_NG = _N // _ZR


def _sc_mesh():
    return plsc.VectorSubcoreMesh(core_axis_name="c", subcore_axis_name="s")


def _worker_plan(ncht):
    cpw = ncht // _NW
    nextra = ncht - cpw * _NW
    return cpw, _NW - nextra


def _gather_half(x, src3, dst3, c0):
    cpw, wextra = _worker_plan(_HALF)
    triples = cpw // 3
    leftover = cpw - triples * 3

    @functools.partial(
        pl.kernel,
        out_type=(jax.ShapeDtypeStruct((_EH, _D), jnp.float32),
                  jax.ShapeDtypeStruct((_EH, _D), jnp.float32)),
        mesh=_sc_mesh(),
        scratch_types=[
            pltpu.VMEM((cpw + 1, 1, _CH), jnp.int32),
            pltpu.VMEM((cpw + 1, 1, _CH), jnp.int32),
            pltpu.VMEM((3, _CH, _D), jnp.float32),
            pltpu.VMEM((3, _CH, _D), jnp.float32),
            pltpu.SemaphoreType.DMA,
            pltpu.SemaphoreType.DMA,
            pltpu.SemaphoreType.DMA,
            pltpu.SemaphoreType.DMA,
            pltpu.SemaphoreType.DMA,
            pltpu.SemaphoreType.DMA,
        ],
    )
    def k(x_hbm, src_hbm, dst_hbm, xs_hbm, xd_hbm,
          isv, idv, rs, rd, sg0, sg1, sg2, sw0, sw1, sw2):
        cid = lax.axis_index("c")
        sid = lax.axis_index("s")
        wid = sid * 2 + cid
        cb = jnp.where(wid < wextra, cpw * wid,
                       cpw * wextra + (cpw + 1) * (wid - wextra))
        extra = wid >= wextra
        semg = (sg0, sg1, sg2)
        semw = (sw0, sw1, sw2)

        pltpu.sync_copy(src_hbm.at[pl.ds(c0 + cb, cpw)], isv.at[pl.ds(0, cpw)])
        pltpu.sync_copy(dst_hbm.at[pl.ds(c0 + cb, cpw)], idv.at[pl.ds(0, cpw)])

        @pl.when(extra)
        def _():
            pltpu.sync_copy(src_hbm.at[pl.ds(c0 + cb + cpw, 1)],
                            isv.at[pl.ds(cpw, 1)])
            pltpu.sync_copy(dst_hbm.at[pl.ds(c0 + cb + cpw, 1)],
                            idv.at[pl.ds(cpw, 1)])

        def drain_wb(k_):
            pltpu.make_async_copy(rs.at[k_], xs_hbm.at[pl.ds(0, _CH)],
                                  semw[k_]).wait()
            pltpu.make_async_copy(rd.at[k_], xd_hbm.at[pl.ds(0, _CH)],
                                  semw[k_]).wait()

        def gath(t, k_):
            gs = pltpu.async_copy(x_hbm.at[isv.at[t, 0]], rs.at[k_], semg[k_])
            gd = pltpu.async_copy(x_hbm.at[idv.at[t, 0]], rd.at[k_], semg[k_])
            return gs, gd

        def wback(t, k_):
            off = pl.multiple_of((cb + t) * _CH, 8)
            pltpu.async_copy(rs.at[k_], xs_hbm.at[pl.ds(off, _CH)], semw[k_])
            pltpu.async_copy(rd.at[k_], xd_hbm.at[pl.ds(off, _CH)], semw[k_])

        def triple(q, carry):
            t0 = q * 3
            hs = []
            for k_ in range(3):
                @pl.when(q > 0)
                def _(k_=k_):
                    drain_wb(k_)
                hs.append(gath(t0 + k_, k_))
            for k_ in range(3):
                gs, gd = hs[k_]
                gs.wait()
                gd.wait()
                wback(t0 + k_, k_)
            return carry

        lax.fori_loop(0, triples, triple, 0)

        for j in range(leftover):
            t = triples * 3 + j
            drain_wb(j)
            gs, gd = gath(t, j)
            gs.wait()
            gd.wait()
            wback(t, j)

        @pl.when(extra)
        def _():
            k_ = leftover
            drain_wb(k_)
            gs, gd = gath(cpw, k_)
            gs.wait()
            gd.wait()
            wback(cpw, k_)

        for k_ in range(3):
            drain_wb(k_)

    return k(x, src3, dst3)


def _scatter_half(e, dst3, c0):
    cpw, wextra = _worker_plan(_HALF)
    pairs = (cpw - 1) // 2

    @functools.partial(
        pl.kernel,
        out_type=jax.ShapeDtypeStruct((2, _N, _D), jnp.float32),
        mesh=_sc_mesh(),
        scratch_types=[
            pltpu.VMEM((cpw + 1, 1, _CH), jnp.int32),
            pltpu.VMEM((2, _CH, _D), jnp.float32),
            pltpu.VMEM((_ZR, _D), jnp.float32),
            pltpu.VMEM_SHARED((_N, _D), jnp.float32),
            pltpu.SemaphoreType.DMA,
            pltpu.SemaphoreType.DMA,
            pltpu.SemaphoreType.DMA,
            pltpu.SemaphoreType.DMA,
        ],
    )
    def k(e_hbm, dst_hbm, out_hbm, idv, rw, zbuf, acc,
          sl0, sl1, ss0, ss1):
        cid = lax.axis_index("c")
        sid = lax.axis_index("s")
        wid = sid * 2 + cid
        cb = jnp.where(wid < wextra, cpw * wid,
                       cpw * wextra + (cpw + 1) * (wid - wextra))
        extra = wid >= wextra
        seml = (sl0, sl1)
        sems = (ss0, ss1)

        zero = jnp.zeros((16,), jnp.float32)
        for r in range(_ZR):
            for l in range(_D // 16):
                zbuf[r, pl.ds(16 * l, 16)] = zero
        ngrp = jnp.where(sid < _NG - 16 * (_NG // 16), _NG // 16 + 1, _NG // 16)

        def zfill(gk, carry):
            roff = pl.multiple_of((sid + gk * 16) * _ZR, 8)
            pltpu.sync_copy(zbuf, acc.at[pl.ds(roff, _ZR)])
            return carry

        lax.fori_loop(0, ngrp, zfill, 0)

        pltpu.sync_copy(dst_hbm.at[pl.ds(c0 + cb, cpw)], idv.at[pl.ds(0, cpw)])

        @pl.when(extra)
        def _():
            pltpu.sync_copy(dst_hbm.at[pl.ds(c0 + cb + cpw, 1)],
                            idv.at[pl.ds(cpw, 1)])

        plsc.subcore_barrier()

        def drain_add(k_):
            pltpu.make_async_copy(rw.at[k_], acc.at[pl.ds(0, _CH)],
                                  sems[k_]).wait()

        def load(t, k_):
            off = pl.multiple_of((cb + t) * _CH, 8)
            return pltpu.async_copy(e_hbm.at[pl.ds(off, _CH)], rw.at[k_],
                                    seml[k_])

        def spair(p, carry):
            t0 = p * 2
            hs = []
            for k_ in range(2):
                @pl.when(p > 0)
                def _(k_=k_):
                    drain_add(k_)
                hs.append(load(t0 + k_, k_))
            for k_ in range(2):
                hs[k_].wait()
                pltpu.async_copy(rw.at[k_], acc.at[idv.at[t0 + k_, 0]],
                                 sems[k_], add=True)
            return carry

        lax.fori_loop(0, pairs, spair, 0)

        drain_add(0)
        l0 = load(cpw - 1, 0)
        l0.wait()
        pltpu.async_copy(rw.at[0], acc.at[idv.at[cpw - 1, 0]],
                         sems[0], add=True)

        @pl.when(extra)
        def _():
            drain_add(1)
            l1 = load(cpw, 1)
            l1.wait()
            pltpu.async_copy(rw.at[1], acc.at[idv.at[cpw, 0]],
                             sems[1], add=True)

        drain_add(0)
        drain_add(1)
        plsc.subcore_barrier()

        def wback(gk, carry):
            roff = pl.multiple_of((sid + gk * 16) * _ZR, 8)
            pltpu.sync_copy(acc.at[pl.ds(roff, _ZR)],
                            out_hbm.at[cid, pl.ds(roff, _ZR)])
            return carry

        lax.fori_loop(0, ngrp, wback, 0)

    return k(e, dst3)


def _mlp_pallas(groups, w1s, b1, w2, b2, w3, b3, ln_g, ln_b, resid_gi, br,
                in_offs=None, n_rows=None):
    flat = [a for g in groups for a in g]
    sizes = [len(g) for g in groups]
    n = n_rows if n_rows is not None else flat[0].shape[0]
    num_in = len(flat)
    ng = len(w1s)
    has_ln = ln_g is not None
    offs = in_offs if in_offs is not None else [0] * num_in

    def row_spec(d, off):
        return pl.BlockSpec((br, d), lambda i, o=off: (i + o, 0))

    full_spec = lambda s: pl.BlockSpec(s, lambda i: (0, 0))

    in_specs = [row_spec(a.shape[1], o) for a, o in zip(flat, offs)]
    in_specs += [full_spec(w.shape) for w in w1s]
    b1r = b1.reshape(1, _D)
    b2r = b2.reshape(1, _D)
    b3r = b3.reshape(1, _D)
    extras = [b1r, w2, b2r, w3, b3r]
    in_specs += [full_spec(b1r.shape), full_spec(w2.shape),
                 full_spec(b2r.shape), full_spec(w3.shape),
                 full_spec(b3r.shape)]
    if has_ln:
        extras += [ln_g.reshape(1, _D), ln_b.reshape(1, _D)]
        in_specs += [full_spec((1, _D)), full_spec((1, _D))]

    def body(*refs):
        irefs = refs[:num_in]
        wrefs = refs[num_in:num_in + ng]
        rest = refs[num_in + ng:]
        b1_r, w2_r, b2_r, w3_r, b3_r = rest[:5]
        out_ref = refs[-1]
        pos = 0
        h = None
        res = None
        for gi, sz in enumerate(sizes):
            xg = irefs[pos][...]
            for j in range(1, sz):
                xg = xg + irefs[pos + j][...]
            t = jnp.dot(xg, wrefs[gi][...], preferred_element_type=jnp.float32)
            h = t if h is None else h + t
            if resid_gi is not None and gi == resid_gi:
                res = xg
            pos += sz
        h = jax.nn.relu(h + b1_r[...])
        h = jax.nn.relu(jnp.dot(h, w2_r[...], preferred_element_type=jnp.float32) + b2_r[...])
        h = jnp.dot(h, w3_r[...], preferred_element_type=jnp.float32) + b3_r[...]
        if has_ln:
            g_r, bb_r = rest[5], rest[6]
            mu = jnp.mean(h, axis=-1, keepdims=True)
            var = jnp.mean((h - mu) * (h - mu), axis=-1, keepdims=True)
            h = (h - mu) * lax.rsqrt(var + 1e-5) * g_r[...] + bb_r[...]
        if res is not None:
            h = res + h
        out_ref[...] = h

    return pl.pallas_call(
        body,
        grid=(n // br,),
        in_specs=in_specs,
        out_specs=pl.BlockSpec((br, _D), lambda i: (i, 0)),
        out_shape=jax.ShapeDtypeStruct((n, _D), jnp.float32),
    )(*flat, *w1s, *extras)


def kernel(init_position, time_vector, node_type, edge_index, edge_features, params):
    p = params
    onehot = jax.nn.one_hot(node_type, 9, dtype=jnp.float32)
    nf = jnp.concatenate(
        [init_position, time_vector[:, None], onehot,
         jnp.zeros((_N, 4), jnp.float32)], axis=1)
    src3 = edge_index[0].reshape(_NCHUNK, 1, _CH)
    dst3 = edge_index[1].reshape(_NCHUNK, 1, _CH)

    ne = p["node_enc"]
    w1n = jnp.concatenate([ne["W"][0], jnp.zeros((4, _D), jnp.float32)], axis=0)
    x = _mlp_pallas([[nf]], [w1n], ne["b"][0], ne["W"][1], ne["b"][1],
                    ne["W"][2], ne["b"][2], ne["ln_g"], ne["ln_b"],
                    resid_gi=None, br=1000)

    ee = p["edge_enc"]
    e_full = _mlp_pallas([[edge_features]], [ee["W"][0]], ee["b"][0],
                         ee["W"][1], ee["b"][1], ee["W"][2], ee["b"][2],
                         ee["ln_g"], ee["ln_b"], resid_gi=None, br=1000)
    eh = [None, None]

    for si, sp in enumerate(p["proc"]):
        ew = sp["edge"]
        wa = ew["W"][0][:_D]
        wb = ew["W"][0][_D:2 * _D]
        wc = ew["W"][0][2 * _D:]
        parts = []
        new_eh = [None, None]
        for h in range(2):
            c0 = h * _HALF
            xs, xd = _gather_half(x, src3, dst3, c0)
            if si == 0:
                e_in, off = e_full, h * (_EH // 1000)
            else:
                e_in, off = eh[h], 0
            new_eh[h] = _mlp_pallas(
                [[xs], [xd], [e_in]], [wa, wb, wc], ew["b"][0],
                ew["W"][1], ew["b"][1], ew["W"][2], ew["b"][2],
                ew["ln_g"], ew["ln_b"], resid_gi=2, br=1000,
                in_offs=[0, 0, off], n_rows=_EH)
            parts.append(_scatter_half(new_eh[h], dst3, c0))
        eh = new_eh
        nw = sp["node"]
        na = nw["W"][0][:_D]
        nb = nw["W"][0][_D:]
        x = _mlp_pallas(
            [[x], [parts[0][0], parts[0][1], parts[1][0], parts[1][1]]],
            [na, nb], nw["b"][0], nw["W"][1], nw["b"][1], nw["W"][2],
            nw["b"][2], nw["ln_g"], nw["ln_b"], resid_gi=0, br=1000)

    dp = p["dec"]
    w3d = jnp.pad(dp["W"][2], ((0, 0), (0, _D - 2)))
    b3d = jnp.pad(dp["b"][2], (0, _D - 2))
    out = _mlp_pallas([[x]], [dp["W"][0]], dp["b"][0], dp["W"][1], dp["b"][1],
                      w3d, b3d, None, None, resid_gi=None, br=1000)
    return init_position + out[:, :2]
